# baseline (speedup 1.0000x reference)
import functools

import jax
import jax.numpy as jnp
from jax import lax
from jax.experimental import pallas as pl
from jax.experimental.pallas import tpu as pltpu
from jax.experimental.pallas import tpu_sc as plsc

_K_PERCENT = 0.05
_NBINS = 2048
_NC = 2
_NS = 16
_NW = _NC * _NS


def _monotone(v):
    b = lax.bitcast_convert_type(v, jnp.int32)
    return b ^ ((b >> 31) & jnp.int32(0x7FFFFFFF))


def _sc_pass1_body(n, chunk, rows_per, x_hbm, cnt_hbm, sum_hbm, buf, hcnt, hsum):
    wid = lax.axis_index("s") * _NC + lax.axis_index("c")
    zeros = jnp.zeros((16,), jnp.float32)
    ones = jnp.full((16,), 1.0, jnp.float32)

    def do_row(r, _):
        row = wid * rows_per + r

        def zero(j, _):
            hcnt[pl.ds(j * 16, 16)] = zeros
            hsum[pl.ds(j * 16, 16)] = zeros
            return 0

        lax.fori_loop(0, _NBINS // 16, zero, 0)

        def do_chunk(c, _):
            pltpu.sync_copy(x_hbm.at[pl.ds(row * n + c * chunk, chunk)], buf)

            def step(j, _):
                v = buf[pl.ds(j * 16, 16)]
                b1 = (_monotone(v) >> 21) + 1024
                plsc.addupdate_scatter(hcnt, [b1], ones)
                plsc.addupdate_scatter(hsum, [b1], v)
                return 0

            lax.fori_loop(0, chunk // 16, step, 0, unroll=8)
            return 0

        lax.fori_loop(0, n // chunk, do_chunk, 0)
        pltpu.sync_copy(hcnt, cnt_hbm.at[pl.ds(row * _NBINS, _NBINS)])
        pltpu.sync_copy(hsum, sum_hbm.at[pl.ds(row * _NBINS, _NBINS)])
        return 0

    lax.fori_loop(0, rows_per, do_row, 0)


def _sc_pass2_body(n, chunk, rows_per, x_hbm, sel_hbm, cnt_hbm, sum_hbm,
                   buf, hcnt, hsum, selbuf):
    wid = lax.axis_index("s") * _NC + lax.axis_index("c")
    zeros = jnp.zeros((16,), jnp.float32)
    ones = jnp.full((16,), 1.0, jnp.float32)

    def do_row(r, _):
        row = wid * rows_per + r
        pltpu.sync_copy(sel_hbm.at[pl.ds(row * 16, 16)], selbuf)

        def zero(j, _):
            hcnt[pl.ds(j * 16, 16)] = zeros
            hsum[pl.ds(j * 16, 16)] = zeros
            return 0

        lax.fori_loop(0, _NBINS // 16, zero, 0)
        selv = selbuf[...]

        def do_chunk(c, _):
            pltpu.sync_copy(x_hbm.at[pl.ds(row * n + c * chunk, chunk)], buf)

            def step(j, _):
                v = buf[pl.ds(j * 16, 16)]
                m = _monotone(v)
                b1 = (m >> 21) + 1024
                match = b1 == selv
                b2 = (m >> 10) & jnp.int32(0x7FF)
                plsc.addupdate_scatter(hcnt, [b2], ones, mask=match)
                plsc.addupdate_scatter(hsum, [b2], v, mask=match)
                return 0

            lax.fori_loop(0, chunk // 16, step, 0, unroll=8)
            return 0

        lax.fori_loop(0, n // chunk, do_chunk, 0)
        pltpu.sync_copy(hcnt, cnt_hbm.at[pl.ds(row * _NBINS, _NBINS)])
        pltpu.sync_copy(hsum, sum_hbm.at[pl.ds(row * _NBINS, _NBINS)])
        return 0

    lax.fori_loop(0, rows_per, do_row, 0)


def _suffix(mat, nbins):
    jj = lax.broadcasted_iota(jnp.int32, (nbins, nbins), 0)
    kk = lax.broadcasted_iota(jnp.int32, (nbins, nbins), 1)
    tri = jnp.where(jj > kk, 1.0, 0.0)
    return jnp.dot(mat, tri, preferred_element_type=jnp.float32)


def _select_body(cnt_ref, sum_ref, sel_ref, meta_ref, *, k, nbins, rb):
    cnt = cnt_ref[...]
    sm = sum_ref[...]
    cex = _suffix(cnt, nbins)
    sex = _suffix(sm, nbins)
    kf = jnp.float32(k)
    mask = (cex < kf) & (cex + cnt >= kf)
    colf = lax.broadcasted_iota(jnp.int32, (rb, nbins), 1).astype(jnp.float32)
    sel = jnp.sum(jnp.where(mask, colf, 0.0), axis=1, keepdims=True)
    c_ab = jnp.sum(jnp.where(mask, cex, 0.0), axis=1, keepdims=True)
    s_ab = jnp.sum(jnp.where(mask, sex, 0.0), axis=1, keepdims=True)
    sel_ref[...] = jnp.broadcast_to(sel, (rb, 16)).astype(jnp.int32)
    lane = lax.broadcasted_iota(jnp.int32, (rb, 128), 1)
    meta_ref[...] = jnp.where(
        lane == 0, jnp.broadcast_to(c_ab, (rb, 128)),
        jnp.where(lane == 1, jnp.broadcast_to(s_ab, (rb, 128)), 0.0))


def _final_body(cnt_ref, sum_ref, meta_ref, peak_ref, *, k, nbins, rb):
    cnt = cnt_ref[...]
    sm = sum_ref[...]
    meta = meta_ref[...]
    c_ab1 = meta[:, 0:1]
    s_ab1 = meta[:, 1:2]
    r1 = jnp.float32(k) - c_ab1
    cex = _suffix(cnt, nbins)
    sex = _suffix(sm, nbins)
    mask = (cex < r1) & (cex + cnt >= r1)
    c_ab2 = jnp.sum(jnp.where(mask, cex, 0.0), axis=1, keepdims=True)
    s_ab2 = jnp.sum(jnp.where(mask, sex, 0.0), axis=1, keepdims=True)
    cstar = jnp.sum(jnp.where(mask, cnt, 0.0), axis=1, keepdims=True)
    sstar = jnp.sum(jnp.where(mask, sm, 0.0), axis=1, keepdims=True)
    r = r1 - c_ab2
    mu = sstar / jnp.maximum(cstar, 1.0)
    topk_sum = s_ab1 + s_ab2 + r * mu
    peak_ref[...] = jnp.broadcast_to(topk_sum / jnp.float32(k), (rb, 128))


def _loss_body(p_ref, s_ref, t_ref, o_ref, *, nb, nc):
    z = p_ref[...]
    s = s_ref[0, 0]
    sp = jnp.maximum(s, 0.0) + jnp.log(1.0 + jnp.exp(-jnp.abs(s)))
    z = z * sp
    m = jnp.max(z, axis=1, keepdims=True)
    lse = m + jnp.log(jnp.sum(jnp.exp(z - m), axis=1, keepdims=True))
    lp = z - lse
    cols = lax.broadcasted_iota(jnp.int32, (nb, nc), 1)
    sel = jnp.sum(jnp.where(cols == t_ref[...], lp, 0.0)) / nb
    o_ref[...] = jnp.full((8, 128), -sel, dtype=jnp.float32)


def kernel(inputs, scale, targets_class):
    B, C, H, W = inputs.shape
    n = H * W
    k = max(1, int(n * _K_PERCENT))
    rows = B * C
    assert rows % _NW == 0
    rows_per = rows // _NW
    chunk = 36864
    assert n % chunk == 0
    x1d = inputs.reshape(-1)

    mesh = plsc.VectorSubcoreMesh(core_axis_name="c", subcore_axis_name="s")
    hist_ty = jax.ShapeDtypeStruct((rows * _NBINS,), jnp.float32)

    cnt1, sum1 = pl.kernel(
        functools.partial(_sc_pass1_body, n, chunk, rows_per),
        mesh=mesh,
        compiler_params=pltpu.CompilerParams(needs_layout_passes=False),
        out_type=[hist_ty, hist_ty],
        scratch_types=[
            pltpu.VMEM((chunk,), jnp.float32),
            pltpu.VMEM((_NBINS,), jnp.float32),
            pltpu.VMEM((_NBINS,), jnp.float32),
        ],
    )(x1d)

    rb = 128
    sel16, meta = pl.pallas_call(
        functools.partial(_select_body, k=k, nbins=_NBINS, rb=rb),
        grid=(rows // rb,),
        in_specs=[
            pl.BlockSpec((rb, _NBINS), lambda i: (i, 0)),
            pl.BlockSpec((rb, _NBINS), lambda i: (i, 0)),
        ],
        out_specs=[
            pl.BlockSpec((rb, 16), lambda i: (i, 0)),
            pl.BlockSpec((rb, 128), lambda i: (i, 0)),
        ],
        out_shape=[
            jax.ShapeDtypeStruct((rows, 16), jnp.int32),
            jax.ShapeDtypeStruct((rows, 128), jnp.float32),
        ],
    )(cnt1.reshape(rows, _NBINS), sum1.reshape(rows, _NBINS))

    cnt2, sum2 = pl.kernel(
        functools.partial(_sc_pass2_body, n, chunk, rows_per),
        mesh=mesh,
        compiler_params=pltpu.CompilerParams(needs_layout_passes=False),
        out_type=[hist_ty, hist_ty],
        scratch_types=[
            pltpu.VMEM((chunk,), jnp.float32),
            pltpu.VMEM((_NBINS,), jnp.float32),
            pltpu.VMEM((_NBINS,), jnp.float32),
            pltpu.VMEM((16,), jnp.int32),
        ],
    )(x1d, sel16.reshape(-1))

    peaks = pl.pallas_call(
        functools.partial(_final_body, k=k, nbins=_NBINS, rb=rb),
        grid=(rows // rb,),
        in_specs=[
            pl.BlockSpec((rb, _NBINS), lambda i: (i, 0)),
            pl.BlockSpec((rb, _NBINS), lambda i: (i, 0)),
            pl.BlockSpec((rb, 128), lambda i: (i, 0)),
        ],
        out_specs=pl.BlockSpec((rb, 128), lambda i: (i, 0)),
        out_shape=jax.ShapeDtypeStruct((rows, 128), jnp.float32),
    )(cnt2.reshape(rows, _NBINS), sum2.reshape(rows, _NBINS), meta)

    peak_logits = peaks[:, 0].reshape(B, C)
    scale2d = scale.reshape(1, 1).astype(jnp.float32)
    tgt = targets_class.astype(jnp.int32).reshape(B, 1)

    loss = pl.pallas_call(
        functools.partial(_loss_body, nb=B, nc=C),
        in_specs=[
            pl.BlockSpec((B, C), lambda: (0, 0)),
            pl.BlockSpec((1, 1), lambda: (0, 0)),
            pl.BlockSpec((B, 1), lambda: (0, 0)),
        ],
        out_specs=pl.BlockSpec((8, 128), lambda: (0, 0)),
        out_shape=jax.ShapeDtypeStruct((8, 128), jnp.float32),
    )(peak_logits, scale2d, tgt)

    return loss[0, 0]


# speedup vs baseline: 1.9641x; 1.9641x over previous
import functools

import jax
import jax.numpy as jnp
from jax import lax
from jax.experimental import pallas as pl
from jax.experimental.pallas import tpu as pltpu
from jax.experimental.pallas import tpu_sc as plsc

_K_PERCENT = 0.05
_NBINS = 2048
_NC = 2
_NS = 16
_NW = _NC * _NS


def _monotone(v):
    b = lax.bitcast_convert_type(v, jnp.int32)
    return b ^ ((b >> 31) & jnp.int32(0x7FFFFFFF))


_NREP = 4


def _sc_pass1_body(n, chunk, rows_per, x_hbm, cnt_hbm, sum_hbm, buf, *hists):
    hcnts = hists[:_NREP]
    hsums = hists[_NREP:]
    wid = lax.axis_index("s") * _NC + lax.axis_index("c")
    zeros = jnp.zeros((16,), jnp.float32)
    ones = jnp.full((16,), 1.0, jnp.float32)
    group = 16 * _NREP

    def do_row(r, _):
        row = wid * rows_per + r

        def zero(j, _):
            for h in hists:
                h[pl.ds(j * 16, 16)] = zeros
            return 0

        lax.fori_loop(0, _NBINS // 16, zero, 0)

        def do_chunk(c, _):
            pltpu.sync_copy(x_hbm.at[pl.ds(row * n + c * chunk, chunk)], buf)

            def step(j, _):
                base = j * group
                vs, idxs = [], []
                for t in range(_NREP):
                    v = buf[pl.ds(base + t * 16, 16)]
                    vs.append(v)
                    idxs.append((_monotone(v) >> 21) + 1024)
                for t in range(_NREP):
                    plsc.addupdate_scatter(hcnts[t], [idxs[t]], ones)
                    plsc.addupdate_scatter(hsums[t], [idxs[t]], vs[t])
                return 0

            lax.fori_loop(0, chunk // group, step, 0, unroll=4)
            return 0

        lax.fori_loop(0, n // chunk, do_chunk, 0)

        def merge(j, _):
            o = j * 16
            c = hcnts[0][pl.ds(o, 16)]
            s = hsums[0][pl.ds(o, 16)]
            for t in range(1, _NREP):
                c = c + hcnts[t][pl.ds(o, 16)]
                s = s + hsums[t][pl.ds(o, 16)]
            hcnts[0][pl.ds(o, 16)] = c
            hsums[0][pl.ds(o, 16)] = s
            return 0

        lax.fori_loop(0, _NBINS // 16, merge, 0)
        pltpu.sync_copy(hcnts[0], cnt_hbm.at[pl.ds(row * _NBINS, _NBINS)])
        pltpu.sync_copy(hsums[0], sum_hbm.at[pl.ds(row * _NBINS, _NBINS)])
        return 0

    lax.fori_loop(0, rows_per, do_row, 0)


def _sc_pass2_body(n, chunk, rows_per, x_hbm, sel_hbm, cnt_hbm, sum_hbm,
                   buf, selbuf, *hists):
    hcnts = hists[:_NREP]
    hsums = hists[_NREP:]
    wid = lax.axis_index("s") * _NC + lax.axis_index("c")
    zeros = jnp.zeros((16,), jnp.float32)
    ones = jnp.full((16,), 1.0, jnp.float32)
    group = 16 * _NREP

    def do_row(r, _):
        row = wid * rows_per + r
        pltpu.sync_copy(sel_hbm.at[pl.ds(row * 16, 16)], selbuf)

        def zero(j, _):
            for h in hists:
                h[pl.ds(j * 16, 16)] = zeros
            return 0

        lax.fori_loop(0, _NBINS // 16, zero, 0)
        selv = selbuf[...]

        def do_chunk(c, _):
            pltpu.sync_copy(x_hbm.at[pl.ds(row * n + c * chunk, chunk)], buf)

            def step(j, _):
                base = j * group
                vs, idxs, masks = [], [], []
                for t in range(_NREP):
                    v = buf[pl.ds(base + t * 16, 16)]
                    m = _monotone(v)
                    vs.append(v)
                    masks.append(((m >> 21) + 1024) == selv)
                    idxs.append((m >> 10) & jnp.int32(0x7FF))
                for t in range(_NREP):
                    plsc.addupdate_scatter(hcnts[t], [idxs[t]], ones, mask=masks[t])
                    plsc.addupdate_scatter(hsums[t], [idxs[t]], vs[t], mask=masks[t])
                return 0

            lax.fori_loop(0, chunk // group, step, 0, unroll=4)
            return 0

        lax.fori_loop(0, n // chunk, do_chunk, 0)

        def merge(j, _):
            o = j * 16
            c = hcnts[0][pl.ds(o, 16)]
            s = hsums[0][pl.ds(o, 16)]
            for t in range(1, _NREP):
                c = c + hcnts[t][pl.ds(o, 16)]
                s = s + hsums[t][pl.ds(o, 16)]
            hcnts[0][pl.ds(o, 16)] = c
            hsums[0][pl.ds(o, 16)] = s
            return 0

        lax.fori_loop(0, _NBINS // 16, merge, 0)
        pltpu.sync_copy(hcnts[0], cnt_hbm.at[pl.ds(row * _NBINS, _NBINS)])
        pltpu.sync_copy(hsums[0], sum_hbm.at[pl.ds(row * _NBINS, _NBINS)])
        return 0

    lax.fori_loop(0, rows_per, do_row, 0)


def _suffix(mat, nbins):
    jj = lax.broadcasted_iota(jnp.int32, (nbins, nbins), 0)
    kk = lax.broadcasted_iota(jnp.int32, (nbins, nbins), 1)
    tri = jnp.where(jj > kk, 1.0, 0.0)
    return jnp.dot(mat, tri, preferred_element_type=jnp.float32)


def _select_body(cnt_ref, sum_ref, sel_ref, meta_ref, *, k, nbins, rb):
    cnt = cnt_ref[...]
    sm = sum_ref[...]
    cex = _suffix(cnt, nbins)
    sex = _suffix(sm, nbins)
    kf = jnp.float32(k)
    mask = (cex < kf) & (cex + cnt >= kf)
    colf = lax.broadcasted_iota(jnp.int32, (rb, nbins), 1).astype(jnp.float32)
    sel = jnp.sum(jnp.where(mask, colf, 0.0), axis=1, keepdims=True)
    c_ab = jnp.sum(jnp.where(mask, cex, 0.0), axis=1, keepdims=True)
    s_ab = jnp.sum(jnp.where(mask, sex, 0.0), axis=1, keepdims=True)
    sel_ref[...] = jnp.broadcast_to(sel, (rb, 16)).astype(jnp.int32)
    lane = lax.broadcasted_iota(jnp.int32, (rb, 128), 1)
    meta_ref[...] = jnp.where(
        lane == 0, jnp.broadcast_to(c_ab, (rb, 128)),
        jnp.where(lane == 1, jnp.broadcast_to(s_ab, (rb, 128)), 0.0))


def _final_body(cnt_ref, sum_ref, meta_ref, peak_ref, *, k, nbins, rb):
    cnt = cnt_ref[...]
    sm = sum_ref[...]
    meta = meta_ref[...]
    c_ab1 = meta[:, 0:1]
    s_ab1 = meta[:, 1:2]
    r1 = jnp.float32(k) - c_ab1
    cex = _suffix(cnt, nbins)
    sex = _suffix(sm, nbins)
    mask = (cex < r1) & (cex + cnt >= r1)
    c_ab2 = jnp.sum(jnp.where(mask, cex, 0.0), axis=1, keepdims=True)
    s_ab2 = jnp.sum(jnp.where(mask, sex, 0.0), axis=1, keepdims=True)
    cstar = jnp.sum(jnp.where(mask, cnt, 0.0), axis=1, keepdims=True)
    sstar = jnp.sum(jnp.where(mask, sm, 0.0), axis=1, keepdims=True)
    r = r1 - c_ab2
    mu = sstar / jnp.maximum(cstar, 1.0)
    topk_sum = s_ab1 + s_ab2 + r * mu
    peak_ref[...] = jnp.broadcast_to(topk_sum / jnp.float32(k), (rb, 128))


def _loss_body(p_ref, s_ref, t_ref, o_ref, *, nb, nc):
    z = p_ref[...]
    s = s_ref[0, 0]
    sp = jnp.maximum(s, 0.0) + jnp.log(1.0 + jnp.exp(-jnp.abs(s)))
    z = z * sp
    m = jnp.max(z, axis=1, keepdims=True)
    lse = m + jnp.log(jnp.sum(jnp.exp(z - m), axis=1, keepdims=True))
    lp = z - lse
    cols = lax.broadcasted_iota(jnp.int32, (nb, nc), 1)
    sel = jnp.sum(jnp.where(cols == t_ref[...], lp, 0.0)) / nb
    o_ref[...] = jnp.full((8, 128), -sel, dtype=jnp.float32)


def kernel(inputs, scale, targets_class):
    B, C, H, W = inputs.shape
    n = H * W
    k = max(1, int(n * _K_PERCENT))
    rows = B * C
    assert rows % _NW == 0
    rows_per = rows // _NW
    chunk = 36864
    assert n % chunk == 0
    x1d = inputs.reshape(-1)

    mesh = plsc.VectorSubcoreMesh(core_axis_name="c", subcore_axis_name="s")
    hist_ty = jax.ShapeDtypeStruct((rows * _NBINS,), jnp.float32)

    cnt1, sum1 = pl.kernel(
        functools.partial(_sc_pass1_body, n, chunk, rows_per),
        mesh=mesh,
        compiler_params=pltpu.CompilerParams(needs_layout_passes=False),
        out_type=[hist_ty, hist_ty],
        scratch_types=(
            [pltpu.VMEM((chunk,), jnp.float32)]
            + [pltpu.VMEM((_NBINS,), jnp.float32) for _ in range(2 * _NREP)]
        ),
    )(x1d)

    rb = 128
    sel16, meta = pl.pallas_call(
        functools.partial(_select_body, k=k, nbins=_NBINS, rb=rb),
        grid=(rows // rb,),
        in_specs=[
            pl.BlockSpec((rb, _NBINS), lambda i: (i, 0)),
            pl.BlockSpec((rb, _NBINS), lambda i: (i, 0)),
        ],
        out_specs=[
            pl.BlockSpec((rb, 16), lambda i: (i, 0)),
            pl.BlockSpec((rb, 128), lambda i: (i, 0)),
        ],
        out_shape=[
            jax.ShapeDtypeStruct((rows, 16), jnp.int32),
            jax.ShapeDtypeStruct((rows, 128), jnp.float32),
        ],
    )(cnt1.reshape(rows, _NBINS), sum1.reshape(rows, _NBINS))

    cnt2, sum2 = pl.kernel(
        functools.partial(_sc_pass2_body, n, chunk, rows_per),
        mesh=mesh,
        compiler_params=pltpu.CompilerParams(needs_layout_passes=False),
        out_type=[hist_ty, hist_ty],
        scratch_types=(
            [pltpu.VMEM((chunk,), jnp.float32), pltpu.VMEM((16,), jnp.int32)]
            + [pltpu.VMEM((_NBINS,), jnp.float32) for _ in range(2 * _NREP)]
        ),
    )(x1d, sel16.reshape(-1))

    peaks = pl.pallas_call(
        functools.partial(_final_body, k=k, nbins=_NBINS, rb=rb),
        grid=(rows // rb,),
        in_specs=[
            pl.BlockSpec((rb, _NBINS), lambda i: (i, 0)),
            pl.BlockSpec((rb, _NBINS), lambda i: (i, 0)),
            pl.BlockSpec((rb, 128), lambda i: (i, 0)),
        ],
        out_specs=pl.BlockSpec((rb, 128), lambda i: (i, 0)),
        out_shape=jax.ShapeDtypeStruct((rows, 128), jnp.float32),
    )(cnt2.reshape(rows, _NBINS), sum2.reshape(rows, _NBINS), meta)

    peak_logits = peaks[:, 0].reshape(B, C)
    scale2d = scale.reshape(1, 1).astype(jnp.float32)
    tgt = targets_class.astype(jnp.int32).reshape(B, 1)

    loss = pl.pallas_call(
        functools.partial(_loss_body, nb=B, nc=C),
        in_specs=[
            pl.BlockSpec((B, C), lambda: (0, 0)),
            pl.BlockSpec((1, 1), lambda: (0, 0)),
            pl.BlockSpec((B, 1), lambda: (0, 0)),
        ],
        out_specs=pl.BlockSpec((8, 128), lambda: (0, 0)),
        out_shape=jax.ShapeDtypeStruct((8, 128), jnp.float32),
    )(peak_logits, scale2d, tgt)

    return loss[0, 0]
